# baseline (speedup 1.0000x reference)
import functools

import jax
import jax.numpy as jnp
from jax import lax
from jax.experimental import pallas as pl
from jax.experimental.pallas import tpu as pltpu
from jax.experimental.pallas import tpu_sc as plsc

N = 10000
NPAD = 10240
E = 320000
EPAD = 327680
DF = 128
DH = 128
DHH = DH // 2
NCLS = 7
DC = 16
CHUNK = 128
ROWS_PER_TILE = NPAD // 32

_mesh = plsc.VectorSubcoreMesh(core_axis_name="c", subcore_axis_name="s")


_DEG_STAGE = 2048
_EDGES_PER_TILE_DEG = EPAD // 32
_DEG_NODES = NPAD // 16


@functools.partial(
    pl.kernel,
    out_type=jax.ShapeDtypeStruct((2, NPAD), jnp.float32),
    mesh=_mesh,
    compiler_params=pltpu.CompilerParams(needs_layout_passes=False),
    scratch_types=[
        pltpu.VMEM((NPAD,), jnp.float32),
        pltpu.VMEM((_DEG_STAGE,), jnp.int32),
        pltpu.VMEM((16 * _DEG_NODES,), jnp.float32),
        pltpu.VMEM((_DEG_NODES,), jnp.float32),
        pltpu.VMEM_SHARED((16 * NPAD,), jnp.float32),
    ],
)
def _deg_kernel(dst_hbm, zeros_hbm, deg_hbm, hist, idxbuf, red, outbuf, shared):
    c = lax.axis_index("c")
    s = lax.axis_index("s")
    pltpu.sync_copy(zeros_hbm, hist)
    ones = jnp.ones((16,), jnp.float32)
    ebase = c * (EPAD // 2) + s * _EDGES_PER_TILE_DEG

    @pl.loop(0, _EDGES_PER_TILE_DEG // _DEG_STAGE)
    def _chunks(k):
        off = pl.multiple_of(ebase + k * _DEG_STAGE, 8)
        pltpu.sync_copy(dst_hbm.at[pl.ds(off, _DEG_STAGE)], idxbuf)

        @pl.loop(0, _DEG_STAGE // 16)
        def _inner(j):
            idx = idxbuf[pl.ds(j * 16, 16)]
            plsc.addupdate_scatter(hist, [idx], ones)

    sbase = pl.multiple_of(s * NPAD, 8)
    pltpu.sync_copy(hist, shared.at[pl.ds(sbase, NPAD)])
    plsc.subcore_barrier()
    nbase = pl.multiple_of(s * _DEG_NODES, 8)
    for r in range(16):
        pltpu.sync_copy(
            shared.at[pl.ds(pl.multiple_of(r * NPAD + nbase, 8), _DEG_NODES)],
            red.at[pl.ds(r * _DEG_NODES, _DEG_NODES)])

    @pl.loop(0, _DEG_NODES // 16)
    def _red(i):
        acc = jnp.zeros((16,), jnp.float32)
        for r in range(16):
            acc = acc + red[pl.ds(r * _DEG_NODES + i * 16, 16)]
        outbuf[pl.ds(i * 16, 16)] = acc

    pltpu.sync_copy(outbuf, deg_hbm.at[c, pl.ds(nbase, _DEG_NODES)])


def _make_agg(D, nbuf, lag, feat_split, group=1):
    etile = (EPAD // 16) if feat_split else (EPAD // 32)
    nrows = etile // CHUNK
    nch = nrows // group
    epc = CHUNK * group
    dump_rows = NPAD // 16
    assert nch % nbuf == 0 and 0 < lag < nbuf and nrows % group == 0

    @functools.partial(
        pl.kernel,
        out_type=jax.ShapeDtypeStruct((2, NPAD, D), jnp.float32),
        mesh=_mesh,
        compiler_params=pltpu.CompilerParams(
            needs_layout_passes=False,
            use_tc_tiling_on_sc=(None if D % 128 == 0 else False)),
        scratch_types=(
            [pltpu.VMEM((epc, D), jnp.float32) for _ in range(nbuf)]
            + [pltpu.VMEM((etile,), jnp.int32) if group > 1
               else pltpu.VMEM((nrows, CHUNK), jnp.int32),
               pltpu.VMEM((nrows, CHUNK), jnp.int32),
               pltpu.VMEM_SHARED((NPAD, D), jnp.float32)]
            + [pltpu.SemaphoreType.DMA for _ in range(2 * nbuf)]
        ),
    )
    def agg(table_hbm, src_hbm, dst_hbm, zrow_hbm, out_hbm, *refs):
        rows = refs[:nbuf]
        sidx, didx, acc = refs[nbuf], refs[nbuf + 1], refs[nbuf + 2]
        gsem = refs[nbuf + 3:nbuf + 3 + nbuf]
        ssem = refs[nbuf + 3 + nbuf:]
        c = lax.axis_index("c")
        s = lax.axis_index("s")
        zbase = pl.multiple_of(s * dump_rows, 8)
        pltpu.sync_copy(zrow_hbm, acc.at[pl.ds(zbase, dump_rows)])
        if feat_split:
            rbase = pl.multiple_of(s * nrows, 8)
            ebase = pl.multiple_of(s * etile, 8)
            pltpu.sync_copy(src_hbm.at[c, pl.ds(rbase, nrows)]
                            if group == 1 else src_hbm.at[c, pl.ds(ebase, etile)],
                            sidx)
            pltpu.sync_copy(dst_hbm.at[pl.ds(rbase, nrows)], didx)
        else:
            rbase = pl.multiple_of(c * (EPAD // 2 // CHUNK) + s * nrows, 8)
            ebase = pl.multiple_of(c * (EPAD // 2) + s * etile, 8)
            pltpu.sync_copy(src_hbm.at[pl.ds(rbase, nrows)]
                            if group == 1 else src_hbm.at[pl.ds(ebase, etile)],
                            sidx)
            pltpu.sync_copy(dst_hbm.at[pl.ds(rbase, nrows)], didx)
        plsc.subcore_barrier()

        def _gi(g):
            if group == 1:
                return sidx.at[g]
            return sidx.at[pl.ds(pl.multiple_of(g * epc, 8), epc)]

        def gather(g, b):
            pltpu.async_copy(table_hbm.at[_gi(g)], rows[b], gsem[b])

        def wait_gather(g, b):
            pltpu.make_async_copy(table_hbm.at[_gi(g)], rows[b], gsem[b]).wait()

        def scatter(g, b):
            for k in range(group):
                pltpu.async_copy(rows[b].at[pl.ds(k * CHUNK, CHUNK)],
                                 acc.at[didx.at[g * group + k]],
                                 ssem[b], add=True)

        def wait_scatter(g, b):
            for k in range(group):
                pltpu.make_async_copy(rows[b].at[pl.ds(k * CHUNK, CHUNK)],
                                      acc.at[didx.at[g * group + k]],
                                      ssem[b]).wait()

        def slot(g, j, refill):
            wait_gather(g, j)
            scatter(g, j)
            h = g - lag
            bh = (j - lag) % nbuf
            wait_scatter(h, bh)
            if refill:
                gather(h + nbuf, bh)

        for g in range(nbuf):
            gather(g, g)
        for g in range(lag):
            wait_gather(g, g)
            scatter(g, g)
        for j in range(lag, nbuf):
            slot(j, j, refill=True)

        @pl.loop(1, nch // nbuf - 1)
        def _ring(it):
            for j in range(nbuf):
                slot(it * nbuf + j, j, refill=True)

        for j in range(nbuf):
            g = nch - nbuf + j
            slot(g, j, refill=(g - lag + nbuf < nch))
        for i in range(lag):
            g = nch - lag + i
            wait_scatter(g, g % nbuf)

        plsc.subcore_barrier()
        pltpu.sync_copy(acc.at[pl.ds(zbase, dump_rows)],
                        out_hbm.at[c, pl.ds(zbase, dump_rows)])

    return agg


_agg_h = _make_agg(DHH, nbuf=5, lag=3, feat_split=True)
_agg_c = _make_agg(DC, nbuf=5, lag=2, feat_split=False, group=8)


_BLK = 1024
_GRID = (NPAD // _BLK,)


def _tc0_body(x_ref, w_ref, h_ref):
    h_ref[...] = jnp.dot(x_ref[...], w_ref[...],
                         preferred_element_type=jnp.float32)


_tc0 = pl.pallas_call(
    _tc0_body,
    grid=_GRID,
    in_specs=[
        pl.BlockSpec((_BLK, DF), lambda i: (i, 0)),
        pl.BlockSpec((DF, DH), lambda i: (0, 0)),
    ],
    out_specs=pl.BlockSpec((_BLK, DH), lambda i: (i, 0)),
    out_shape=jax.ShapeDtypeStruct((NPAD, DH), jnp.float32),
)


def _tc1_body(h_ref, deg_ref, h1s_ref, dinv_ref):
    dinv = lax.rsqrt(deg_ref[0] + deg_ref[1] + 1.0)
    hs = h_ref[...] * dinv[:, None]
    h1s_ref[0] = hs[:, :DHH]
    h1s_ref[1] = hs[:, DHH:]
    dinv_ref[...] = dinv


_tc1 = pl.pallas_call(
    _tc1_body,
    grid=_GRID,
    in_specs=[
        pl.BlockSpec((_BLK, DH), lambda i: (i, 0)),
        pl.BlockSpec((2, _BLK), lambda i: (0, i)),
    ],
    out_specs=[
        pl.BlockSpec((2, _BLK, DHH), lambda i: (0, i, 0)),
        pl.BlockSpec((_BLK,), lambda i: (i,)),
    ],
    out_shape=[
        jax.ShapeDtypeStruct((2, NPAD, DHH), jnp.float32),
        jax.ShapeDtypeStruct((NPAD,), jnp.float32),
    ],
)


def _tc2_body(agg_ref, h1s_ref, dinv_ref, b1_ref, w2_ref, h2s_ref):
    dinv = dinv_ref[...]
    agg = jnp.concatenate([agg_ref[0], agg_ref[1]], axis=1)
    h1s = jnp.concatenate([h1s_ref[0], h1s_ref[1]], axis=1)
    pre = (agg + h1s) * dinv[:, None] + b1_ref[...][None, :]
    out1 = jnp.maximum(pre, 0.0)
    h2 = jnp.dot(out1, w2_ref[...], preferred_element_type=jnp.float32)
    h2s_ref[...] = h2 * dinv[:, None]


_tc2 = pl.pallas_call(
    _tc2_body,
    grid=_GRID,
    in_specs=[
        pl.BlockSpec((2, _BLK, DHH), lambda i: (0, i, 0)),
        pl.BlockSpec((2, _BLK, DHH), lambda i: (0, i, 0)),
        pl.BlockSpec((_BLK,), lambda i: (i,)),
        pl.BlockSpec((DH,), lambda i: (0,)),
        pl.BlockSpec((DH, DC), lambda i: (0, 0)),
    ],
    out_specs=pl.BlockSpec((_BLK, DC), lambda i: (i, 0)),
    out_shape=jax.ShapeDtypeStruct((NPAD, DC), jnp.float32),
)


def _tc3_body(parts_ref, h2s_ref, dinv_ref, b2_ref, out_ref):
    tot = parts_ref[0] + parts_ref[1] + h2s_ref[...]
    out_ref[...] = tot * dinv_ref[...][:, None] + b2_ref[...][None, :]


_tc3 = pl.pallas_call(
    _tc3_body,
    grid=_GRID,
    in_specs=[
        pl.BlockSpec((2, _BLK, DC), lambda i: (0, i, 0)),
        pl.BlockSpec((_BLK, DC), lambda i: (i, 0)),
        pl.BlockSpec((_BLK,), lambda i: (i,)),
        pl.BlockSpec((DC,), lambda i: (0,)),
    ],
    out_specs=pl.BlockSpec((_BLK, DC), lambda i: (i, 0)),
    out_shape=jax.ShapeDtypeStruct((NPAD, DC), jnp.float32),
)


def kernel(x, edge_index, W1, b1, W2, b2):
    src = edge_index[0].astype(jnp.int32)
    dst = edge_index[1].astype(jnp.int32)
    pad = jnp.full((EPAD - E,), N, jnp.int32)
    srcp = jnp.concatenate([src, pad])
    dstp = jnp.concatenate([dst, pad])
    src2d = srcp.reshape(EPAD // CHUNK, CHUNK)
    dst2d = dstp.reshape(EPAD // CHUNK, CHUNK)
    srcb = jnp.stack([src2d, src2d + NPAD])
    xp = jnp.concatenate([x, jnp.zeros((NPAD - N, DF), x.dtype)], axis=0)
    w2p = jnp.concatenate([W2, jnp.zeros((DH, DC - NCLS), W2.dtype)], axis=1)
    b2p = jnp.concatenate([b2, jnp.zeros((DC - NCLS,), b2.dtype)])
    znodes = jnp.zeros((NPAD,), jnp.float32)
    zrow_h = jnp.zeros((NPAD // 16, DHH), jnp.float32)
    zrow_c = jnp.zeros((NPAD // 16, DC), jnp.float32)

    deg = _deg_kernel(dstp, znodes)
    h1 = _tc0(xp, W1)
    h1s2, dinv = _tc1(h1, deg)
    table1 = h1s2.reshape(2 * NPAD, DHH)
    parts1 = _agg_h(table1, srcb, dst2d, zrow_h)
    h2s = _tc2(parts1, h1s2, dinv, b1, w2p)
    parts2 = _agg_c(h2s, srcp, dst2d, zrow_c)
    out = _tc3(parts2, h2s, dinv, b2p)
    return out[:N, :NCLS]


# speedup vs baseline: 1.0083x; 1.0083x over previous
import functools

import jax
import jax.numpy as jnp
from jax import lax
from jax.experimental import pallas as pl
from jax.experimental.pallas import tpu as pltpu
from jax.experimental.pallas import tpu_sc as plsc

N = 10000
NPAD = 10240
E = 320000
EPAD = 327680
DF = 128
DH = 128
DHH = DH // 2
NCLS = 7
DC = 16
CHUNK = 128
ROWS_PER_TILE = NPAD // 32

_mesh = plsc.VectorSubcoreMesh(core_axis_name="c", subcore_axis_name="s")


_DEG_STAGE = 2048
_EDGES_PER_TILE_DEG = EPAD // 32
_DEG_NODES = NPAD // 16


@functools.partial(
    pl.kernel,
    out_type=jax.ShapeDtypeStruct((2, NPAD), jnp.float32),
    mesh=_mesh,
    compiler_params=pltpu.CompilerParams(needs_layout_passes=False),
    scratch_types=[
        pltpu.VMEM((NPAD,), jnp.float32),
        pltpu.VMEM((_DEG_STAGE,), jnp.int32),
        pltpu.VMEM((16 * _DEG_NODES,), jnp.float32),
        pltpu.VMEM((_DEG_NODES,), jnp.float32),
        pltpu.VMEM_SHARED((16 * NPAD,), jnp.float32),
    ],
)
def _deg_kernel(dst_hbm, zeros_hbm, deg_hbm, hist, idxbuf, red, outbuf, shared):
    c = lax.axis_index("c")
    s = lax.axis_index("s")
    pltpu.sync_copy(zeros_hbm, hist)
    ones = jnp.ones((16,), jnp.float32)
    ebase = c * (EPAD // 2) + s * _EDGES_PER_TILE_DEG

    @pl.loop(0, _EDGES_PER_TILE_DEG // _DEG_STAGE)
    def _chunks(k):
        off = pl.multiple_of(ebase + k * _DEG_STAGE, 8)
        pltpu.sync_copy(dst_hbm.at[pl.ds(off, _DEG_STAGE)], idxbuf)

        @pl.loop(0, _DEG_STAGE // 16)
        def _inner(j):
            idx = idxbuf[pl.ds(j * 16, 16)]
            plsc.addupdate_scatter(hist, [idx], ones)

    sbase = pl.multiple_of(s * NPAD, 8)
    pltpu.sync_copy(hist, shared.at[pl.ds(sbase, NPAD)])
    plsc.subcore_barrier()
    nbase = pl.multiple_of(s * _DEG_NODES, 8)
    for r in range(16):
        pltpu.sync_copy(
            shared.at[pl.ds(pl.multiple_of(r * NPAD + nbase, 8), _DEG_NODES)],
            red.at[pl.ds(r * _DEG_NODES, _DEG_NODES)])

    @pl.loop(0, _DEG_NODES // 16)
    def _red(i):
        acc = jnp.zeros((16,), jnp.float32)
        for r in range(16):
            acc = acc + red[pl.ds(r * _DEG_NODES + i * 16, 16)]
        outbuf[pl.ds(i * 16, 16)] = acc

    pltpu.sync_copy(outbuf, deg_hbm.at[c, pl.ds(nbase, _DEG_NODES)])


def _make_agg(D, nbuf, lag, feat_split, group=1):
    etile = (EPAD // 16) if feat_split else (EPAD // 32)
    nrows = etile // CHUNK
    nch = nrows // group
    epc = CHUNK * group
    dump_rows = NPAD // 16
    assert nch % nbuf == 0 and 0 < lag < nbuf and nrows % group == 0

    @functools.partial(
        pl.kernel,
        out_type=jax.ShapeDtypeStruct((2, NPAD, D), jnp.float32),
        mesh=_mesh,
        compiler_params=pltpu.CompilerParams(
            needs_layout_passes=False,
            use_tc_tiling_on_sc=(None if D % 128 == 0 else False)),
        scratch_types=(
            [pltpu.VMEM((epc, D), jnp.float32) for _ in range(nbuf)]
            + [pltpu.VMEM((etile,), jnp.int32) if group > 1
               else pltpu.VMEM((nrows, CHUNK), jnp.int32),
               pltpu.VMEM((nrows, CHUNK), jnp.int32),
               pltpu.VMEM_SHARED((NPAD, D), jnp.float32)]
            + [pltpu.SemaphoreType.DMA for _ in range(2 * nbuf)]
        ),
    )
    def agg(table_hbm, src_hbm, dst_hbm, zrow_hbm, out_hbm, *refs):
        rows = refs[:nbuf]
        sidx, didx, acc = refs[nbuf], refs[nbuf + 1], refs[nbuf + 2]
        gsem = refs[nbuf + 3:nbuf + 3 + nbuf]
        ssem = refs[nbuf + 3 + nbuf:]
        c = lax.axis_index("c")
        s = lax.axis_index("s")
        zbase = pl.multiple_of(s * dump_rows, 8)
        pltpu.sync_copy(zrow_hbm, acc.at[pl.ds(zbase, dump_rows)])
        if feat_split:
            rbase = pl.multiple_of(s * nrows, 8)
            ebase = pl.multiple_of(s * etile, 8)
            pltpu.sync_copy(src_hbm.at[c, pl.ds(rbase, nrows)]
                            if group == 1 else src_hbm.at[c, pl.ds(ebase, etile)],
                            sidx)
            pltpu.sync_copy(dst_hbm.at[pl.ds(rbase, nrows)], didx)
        else:
            rbase = pl.multiple_of(c * (EPAD // 2 // CHUNK) + s * nrows, 8)
            ebase = pl.multiple_of(c * (EPAD // 2) + s * etile, 8)
            pltpu.sync_copy(src_hbm.at[pl.ds(rbase, nrows)]
                            if group == 1 else src_hbm.at[pl.ds(ebase, etile)],
                            sidx)
            pltpu.sync_copy(dst_hbm.at[pl.ds(rbase, nrows)], didx)
        plsc.subcore_barrier()

        def _gi(g):
            if group == 1:
                return sidx.at[g]
            return sidx.at[pl.ds(pl.multiple_of(g * epc, 8), epc)]

        def gather(g, b):
            pltpu.async_copy(table_hbm.at[_gi(g)], rows[b], gsem[b])

        def wait_gather(g, b):
            pltpu.make_async_copy(table_hbm.at[_gi(g)], rows[b], gsem[b]).wait()

        def scatter(g, b):
            for k in range(group):
                pltpu.async_copy(rows[b].at[pl.ds(k * CHUNK, CHUNK)],
                                 acc.at[didx.at[g * group + k]],
                                 ssem[b], add=True)

        def wait_scatter(g, b):
            for k in range(group):
                pltpu.make_async_copy(rows[b].at[pl.ds(k * CHUNK, CHUNK)],
                                      acc.at[didx.at[g * group + k]],
                                      ssem[b]).wait()

        def slot(g, j, refill):
            wait_gather(g, j)
            scatter(g, j)
            h = g - lag
            bh = (j - lag) % nbuf
            wait_scatter(h, bh)
            if refill:
                gather(h + nbuf, bh)

        for g in range(nbuf):
            gather(g, g)
        for g in range(lag):
            wait_gather(g, g)
            scatter(g, g)
        for j in range(lag, nbuf):
            slot(j, j, refill=True)

        @pl.loop(1, nch // nbuf - 1)
        def _ring(it):
            for j in range(nbuf):
                slot(it * nbuf + j, j, refill=True)

        for j in range(nbuf):
            g = nch - nbuf + j
            slot(g, j, refill=(g - lag + nbuf < nch))
        for i in range(lag):
            g = nch - lag + i
            wait_scatter(g, g % nbuf)

        plsc.subcore_barrier()
        pltpu.sync_copy(acc.at[pl.ds(zbase, dump_rows)],
                        out_hbm.at[c, pl.ds(zbase, dump_rows)])

    return agg


_agg_h = _make_agg(DHH, nbuf=5, lag=2, feat_split=True)
_agg_c = _make_agg(DC, nbuf=5, lag=2, feat_split=False, group=8)


_BLK = 1024
_GRID = (NPAD // _BLK,)


def _tc1_body(x_ref, deg_ref, xs_ref, dinv_ref):
    dinv = lax.rsqrt(deg_ref[0] + deg_ref[1] + 1.0)
    xs = x_ref[...] * dinv[:, None]
    xs_ref[0] = xs[:, :DHH]
    xs_ref[1] = xs[:, DHH:]
    dinv_ref[...] = dinv


_tc1 = pl.pallas_call(
    _tc1_body,
    grid=_GRID,
    in_specs=[
        pl.BlockSpec((_BLK, DF), lambda i: (i, 0)),
        pl.BlockSpec((2, _BLK), lambda i: (0, i)),
    ],
    out_specs=[
        pl.BlockSpec((2, _BLK, DHH), lambda i: (0, i, 0)),
        pl.BlockSpec((_BLK,), lambda i: (i,)),
    ],
    out_shape=[
        jax.ShapeDtypeStruct((2, NPAD, DHH), jnp.float32),
        jax.ShapeDtypeStruct((NPAD,), jnp.float32),
    ],
)


def _tc2_body(agg_ref, xs_ref, dinv_ref, w1_ref, b1_ref, w2_ref, h2s_ref):
    dinv = dinv_ref[...]
    agg = jnp.concatenate([agg_ref[0], agg_ref[1]], axis=1)
    xs = jnp.concatenate([xs_ref[0], xs_ref[1]], axis=1)
    xagg = (agg + xs) * dinv[:, None]
    pre = jnp.dot(xagg, w1_ref[...],
                  preferred_element_type=jnp.float32) + b1_ref[...][None, :]
    out1 = jnp.maximum(pre, 0.0)
    h2 = jnp.dot(out1, w2_ref[...], preferred_element_type=jnp.float32)
    h2s_ref[...] = h2 * dinv[:, None]


_tc2 = pl.pallas_call(
    _tc2_body,
    grid=_GRID,
    in_specs=[
        pl.BlockSpec((2, _BLK, DHH), lambda i: (0, i, 0)),
        pl.BlockSpec((2, _BLK, DHH), lambda i: (0, i, 0)),
        pl.BlockSpec((_BLK,), lambda i: (i,)),
        pl.BlockSpec((DF, DH), lambda i: (0, 0)),
        pl.BlockSpec((DH,), lambda i: (0,)),
        pl.BlockSpec((DH, DC), lambda i: (0, 0)),
    ],
    out_specs=pl.BlockSpec((_BLK, DC), lambda i: (i, 0)),
    out_shape=jax.ShapeDtypeStruct((NPAD, DC), jnp.float32),
)


def _tc3_body(parts_ref, h2s_ref, dinv_ref, b2_ref, out_ref):
    tot = parts_ref[0] + parts_ref[1] + h2s_ref[...]
    out_ref[...] = tot * dinv_ref[...][:, None] + b2_ref[...][None, :]


_tc3 = pl.pallas_call(
    _tc3_body,
    grid=_GRID,
    in_specs=[
        pl.BlockSpec((2, _BLK, DC), lambda i: (0, i, 0)),
        pl.BlockSpec((_BLK, DC), lambda i: (i, 0)),
        pl.BlockSpec((_BLK,), lambda i: (i,)),
        pl.BlockSpec((DC,), lambda i: (0,)),
    ],
    out_specs=pl.BlockSpec((_BLK, DC), lambda i: (i, 0)),
    out_shape=jax.ShapeDtypeStruct((NPAD, DC), jnp.float32),
)


def kernel(x, edge_index, W1, b1, W2, b2):
    src = edge_index[0].astype(jnp.int32)
    dst = edge_index[1].astype(jnp.int32)
    pad = jnp.full((EPAD - E,), N, jnp.int32)
    srcp = jnp.concatenate([src, pad])
    dstp = jnp.concatenate([dst, pad])
    src2d = srcp.reshape(EPAD // CHUNK, CHUNK)
    dst2d = dstp.reshape(EPAD // CHUNK, CHUNK)
    srcb = jnp.stack([src2d, src2d + NPAD])
    xp = jnp.concatenate([x, jnp.zeros((NPAD - N, DF), x.dtype)], axis=0)
    w2p = jnp.concatenate([W2, jnp.zeros((DH, DC - NCLS), W2.dtype)], axis=1)
    b2p = jnp.concatenate([b2, jnp.zeros((DC - NCLS,), b2.dtype)])
    znodes = jnp.zeros((NPAD,), jnp.float32)
    zrow_h = jnp.zeros((NPAD // 16, DHH), jnp.float32)
    zrow_c = jnp.zeros((NPAD // 16, DC), jnp.float32)

    deg = _deg_kernel(dstp, znodes)
    xs2, dinv = _tc1(xp, deg)
    table1 = xs2.reshape(2 * NPAD, DHH)
    parts1 = _agg_h(table1, srcb, dst2d, zrow_h)
    h2s = _tc2(parts1, xs2, dinv, W1, b1, w2p)
    parts2 = _agg_c(h2s, srcp, dst2d, zrow_c)
    out = _tc3(parts2, h2s, dinv, b2p)
    return out[:N, :NCLS]


# speedup vs baseline: 1.0162x; 1.0078x over previous
import functools

import jax
import jax.numpy as jnp
from jax import lax
from jax.experimental import pallas as pl
from jax.experimental.pallas import tpu as pltpu
from jax.experimental.pallas import tpu_sc as plsc

N = 10000
NPAD = 10240
E = 320000
EPAD = 327680
DF = 128
DH = 128
DHH = DH // 2
NCLS = 7
DC = 16
CHUNK = 128
_mesh = plsc.VectorSubcoreMesh(core_axis_name="c", subcore_axis_name="s")


_DEG_STAGE = 2048
_EDGES_PER_TILE_DEG = EPAD // 32
_DEG_NODES = NPAD // 16


@functools.partial(
    pl.kernel,
    out_type=jax.ShapeDtypeStruct((2, NPAD), jnp.float32),
    mesh=_mesh,
    compiler_params=pltpu.CompilerParams(needs_layout_passes=False),
    scratch_types=[
        pltpu.VMEM((NPAD,), jnp.float32),
        pltpu.VMEM((_DEG_STAGE,), jnp.int32),
        pltpu.VMEM((16 * _DEG_NODES,), jnp.float32),
        pltpu.VMEM((_DEG_NODES,), jnp.float32),
        pltpu.VMEM_SHARED((16 * NPAD,), jnp.float32),
    ],
)
def _deg_kernel(dst_hbm, zeros_hbm, deg_hbm, hist, idxbuf, red, outbuf, shared):
    c = lax.axis_index("c")
    s = lax.axis_index("s")
    pltpu.sync_copy(zeros_hbm, hist)
    ones = jnp.ones((16,), jnp.float32)
    ebase = c * (EPAD // 2) + s * _EDGES_PER_TILE_DEG

    @pl.loop(0, _EDGES_PER_TILE_DEG // _DEG_STAGE)
    def _chunks(k):
        off = pl.multiple_of(ebase + k * _DEG_STAGE, 8)
        pltpu.sync_copy(dst_hbm.at[pl.ds(off, _DEG_STAGE)], idxbuf)

        @pl.loop(0, _DEG_STAGE // 16)
        def _inner(j):
            idx = idxbuf[pl.ds(j * 16, 16)]
            plsc.addupdate_scatter(hist, [idx], ones)

    sbase = pl.multiple_of(s * NPAD, 8)
    pltpu.sync_copy(hist, shared.at[pl.ds(sbase, NPAD)])
    plsc.subcore_barrier()
    nbase = pl.multiple_of(s * _DEG_NODES, 8)
    for r in range(16):
        pltpu.sync_copy(
            shared.at[pl.ds(pl.multiple_of(r * NPAD + nbase, 8), _DEG_NODES)],
            red.at[pl.ds(r * _DEG_NODES, _DEG_NODES)])

    @pl.loop(0, _DEG_NODES // 16)
    def _red(i):
        acc = jnp.zeros((16,), jnp.float32)
        for r in range(16):
            acc = acc + red[pl.ds(r * _DEG_NODES + i * 16, 16)]
        outbuf[pl.ds(i * 16, 16)] = acc

    pltpu.sync_copy(outbuf, deg_hbm.at[c, pl.ds(nbase, _DEG_NODES)])


def _make_agg(D, nbuf, lag, feat_split, group=1):
    etile = (EPAD // 16) if feat_split else (EPAD // 32)
    nrows = etile // CHUNK
    nch = nrows // group
    epc = CHUNK * group
    dump_rows = NPAD // 16
    assert nch % nbuf == 0 and 0 < lag < nbuf and nrows % group == 0

    @functools.partial(
        pl.kernel,
        out_type=jax.ShapeDtypeStruct((2, NPAD, D), jnp.float32),
        mesh=_mesh,
        compiler_params=pltpu.CompilerParams(
            needs_layout_passes=False,
            use_tc_tiling_on_sc=(None if D % 128 == 0 else False)),
        scratch_types=(
            [pltpu.VMEM((epc, D), jnp.float32) for _ in range(nbuf)]
            + [pltpu.VMEM((etile,), jnp.int32) if group > 1
               else pltpu.VMEM((nrows, CHUNK), jnp.int32),
               pltpu.VMEM((nrows, CHUNK), jnp.int32),
               pltpu.VMEM_SHARED((NPAD, D), jnp.float32)]
            + [pltpu.SemaphoreType.DMA for _ in range(2 * nbuf)]
        ),
    )
    def agg(table_hbm, src_hbm, dst_hbm, zrow_hbm, out_hbm, *refs):
        rows = refs[:nbuf]
        sidx, didx, acc = refs[nbuf], refs[nbuf + 1], refs[nbuf + 2]
        gsem = refs[nbuf + 3:nbuf + 3 + nbuf]
        ssem = refs[nbuf + 3 + nbuf:]
        c = lax.axis_index("c")
        s = lax.axis_index("s")
        zbase = pl.multiple_of(s * dump_rows, 8)
        pltpu.sync_copy(zrow_hbm, acc.at[pl.ds(zbase, dump_rows)])
        if feat_split:
            rbase = pl.multiple_of(s * nrows, 8)
            ebase = pl.multiple_of(s * etile, 8)
            pltpu.sync_copy(src_hbm.at[c, pl.ds(rbase, nrows)]
                            if group == 1 else src_hbm.at[c, pl.ds(ebase, etile)],
                            sidx)
            pltpu.sync_copy(dst_hbm.at[pl.ds(rbase, nrows)], didx)
        else:
            rbase = pl.multiple_of(c * (EPAD // 2 // CHUNK) + s * nrows, 8)
            ebase = pl.multiple_of(c * (EPAD // 2) + s * etile, 8)
            pltpu.sync_copy(src_hbm.at[pl.ds(rbase, nrows)]
                            if group == 1 else src_hbm.at[pl.ds(ebase, etile)],
                            sidx)
            pltpu.sync_copy(dst_hbm.at[pl.ds(rbase, nrows)], didx)
        plsc.subcore_barrier()

        def _gi(g):
            if group == 1:
                return sidx.at[g]
            return sidx.at[pl.ds(pl.multiple_of(g * epc, 8), epc)]

        def gather(g, b):
            pltpu.async_copy(table_hbm.at[_gi(g)], rows[b], gsem[b])

        def wait_gather(g, b):
            pltpu.make_async_copy(table_hbm.at[_gi(g)], rows[b], gsem[b]).wait()

        def scatter(g, b):
            for k in range(group):
                pltpu.async_copy(rows[b].at[pl.ds(k * CHUNK, CHUNK)],
                                 acc.at[didx.at[g * group + k]],
                                 ssem[b], add=True)

        def wait_scatter(g, b):
            for k in range(group):
                pltpu.make_async_copy(rows[b].at[pl.ds(k * CHUNK, CHUNK)],
                                      acc.at[didx.at[g * group + k]],
                                      ssem[b]).wait()

        def slot(g, j, refill):
            wait_gather(g, j)
            scatter(g, j)
            h = g - lag
            bh = (j - lag) % nbuf
            wait_scatter(h, bh)
            if refill:
                gather(h + nbuf, bh)

        for g in range(nbuf):
            gather(g, g)
        for g in range(lag):
            wait_gather(g, g)
            scatter(g, g)
        for j in range(lag, nbuf):
            slot(j, j, refill=True)

        @pl.loop(1, nch // nbuf - 1)
        def _ring(it):
            for j in range(nbuf):
                slot(it * nbuf + j, j, refill=True)

        for j in range(nbuf):
            g = nch - nbuf + j
            slot(g, j, refill=(g - lag + nbuf < nch))
        for i in range(lag):
            g = nch - lag + i
            wait_scatter(g, g % nbuf)

        plsc.subcore_barrier()
        pltpu.sync_copy(acc.at[pl.ds(zbase, dump_rows)],
                        out_hbm.at[c, pl.ds(zbase, dump_rows)])

    return agg


_agg_h = _make_agg(DHH, nbuf=5, lag=2, feat_split=True)
_agg_c = _make_agg(DC, nbuf=5, lag=2, feat_split=False, group=8)


_BLK = 1024
_GRID = (NPAD // _BLK,)


def _tc0_body(x_ref, w_ref, h_ref):
    h_ref[...] = jnp.dot(x_ref[...], w_ref[...],
                         preferred_element_type=jnp.float32)


_tc0 = pl.pallas_call(
    _tc0_body,
    grid=_GRID,
    in_specs=[
        pl.BlockSpec((_BLK, DF), lambda i: (i, 0)),
        pl.BlockSpec((DF, DH), lambda i: (0, 0)),
    ],
    out_specs=pl.BlockSpec((_BLK, DH), lambda i: (i, 0)),
    out_shape=jax.ShapeDtypeStruct((NPAD, DH), jnp.float32),
)


def _tc1_body(h_ref, deg_ref, h1s_ref, dinv_ref):
    dinv = lax.rsqrt(deg_ref[0] + deg_ref[1] + 1.0)
    hs = h_ref[...] * dinv[:, None]
    h1s_ref[0] = hs[:, :DHH]
    h1s_ref[1] = hs[:, DHH:]
    dinv_ref[...] = dinv


_tc1 = pl.pallas_call(
    _tc1_body,
    grid=_GRID,
    in_specs=[
        pl.BlockSpec((_BLK, DH), lambda i: (i, 0)),
        pl.BlockSpec((2, _BLK), lambda i: (0, i)),
    ],
    out_specs=[
        pl.BlockSpec((2, _BLK, DHH), lambda i: (0, i, 0)),
        pl.BlockSpec((_BLK,), lambda i: (i,)),
    ],
    out_shape=[
        jax.ShapeDtypeStruct((2, NPAD, DHH), jnp.float32),
        jax.ShapeDtypeStruct((NPAD,), jnp.float32),
    ],
)


def _tc2_body(agg_ref, h1s_ref, dinv_ref, b1_ref, w2_ref, h2s_ref):
    dinv = dinv_ref[...]
    agg = jnp.concatenate([agg_ref[0], agg_ref[1]], axis=1)
    h1s = jnp.concatenate([h1s_ref[0], h1s_ref[1]], axis=1)
    pre = (agg + h1s) * dinv[:, None] + b1_ref[...][None, :]
    out1 = jnp.maximum(pre, 0.0)
    h2 = jnp.dot(out1, w2_ref[...], preferred_element_type=jnp.float32)
    h2s_ref[...] = h2 * dinv[:, None]


_tc2 = pl.pallas_call(
    _tc2_body,
    grid=_GRID,
    in_specs=[
        pl.BlockSpec((2, _BLK, DHH), lambda i: (0, i, 0)),
        pl.BlockSpec((2, _BLK, DHH), lambda i: (0, i, 0)),
        pl.BlockSpec((_BLK,), lambda i: (i,)),
        pl.BlockSpec((DH,), lambda i: (0,)),
        pl.BlockSpec((DH, DC), lambda i: (0, 0)),
    ],
    out_specs=pl.BlockSpec((_BLK, DC), lambda i: (i, 0)),
    out_shape=jax.ShapeDtypeStruct((NPAD, DC), jnp.float32),
)


def _tc3_body(parts_ref, h2s_ref, dinv_ref, b2_ref, out_ref):
    tot = parts_ref[0] + parts_ref[1] + h2s_ref[...]
    out_ref[...] = tot * dinv_ref[...][:, None] + b2_ref[...][None, :]


_tc3 = pl.pallas_call(
    _tc3_body,
    grid=_GRID,
    in_specs=[
        pl.BlockSpec((2, _BLK, DC), lambda i: (0, i, 0)),
        pl.BlockSpec((_BLK, DC), lambda i: (i, 0)),
        pl.BlockSpec((_BLK,), lambda i: (i,)),
        pl.BlockSpec((DC,), lambda i: (0,)),
    ],
    out_specs=pl.BlockSpec((_BLK, DC), lambda i: (i, 0)),
    out_shape=jax.ShapeDtypeStruct((NPAD, DC), jnp.float32),
)


def kernel(x, edge_index, W1, b1, W2, b2):
    src = edge_index[0].astype(jnp.int32)
    dst = edge_index[1].astype(jnp.int32)
    pad = jnp.full((EPAD - E,), N, jnp.int32)
    srcp = jnp.concatenate([src, pad])
    dstp = jnp.concatenate([dst, pad])
    src2d = srcp.reshape(EPAD // CHUNK, CHUNK)
    dst2d = dstp.reshape(EPAD // CHUNK, CHUNK)
    srcb = jnp.stack([src2d, src2d + NPAD])
    xp = jnp.concatenate([x, jnp.zeros((NPAD - N, DF), x.dtype)], axis=0)
    w2p = jnp.concatenate([W2, jnp.zeros((DH, DC - NCLS), W2.dtype)], axis=1)
    b2p = jnp.concatenate([b2, jnp.zeros((DC - NCLS,), b2.dtype)])
    znodes = jnp.zeros((NPAD,), jnp.float32)
    zrow_h = jnp.zeros((NPAD // 16, DHH), jnp.float32)
    zrow_c = jnp.zeros((NPAD // 16, DC), jnp.float32)

    deg = _deg_kernel(dstp, znodes)
    h1 = _tc0(xp, W1)
    h1s2, dinv = _tc1(h1, deg)
    table1 = h1s2.reshape(2 * NPAD, DHH)
    parts1 = _agg_h(table1, srcb, dst2d, zrow_h)
    h2s = _tc2(parts1, h1s2, dinv, b1, w2p)
    parts2 = _agg_c(h2s, srcp, dst2d, zrow_c)
    out = _tc3(parts2, h2s, dinv, b2p)
    return out[:N, :NCLS]
